# baseline (speedup 1.0000x reference)
import functools

import jax
import jax.numpy as jnp
from jax import lax
from jax.experimental import pallas as pl
from jax.experimental.pallas import tpu as pltpu
from jax.experimental.pallas import tpu_sc as plsc

B, T, D = 4, 8192, 768
K = max(1, T // 10)
EMPH = 1.5
TT = 8192
NT = T // TT


def _k1_body(x_ref, w_ref, b_ref, u_ref, s1_ref, stats_ref, uall_ref):
    bb = pl.program_id(0)
    t = pl.program_id(1)
    xb = x_ref[0]
    wv = w_ref[...]
    e = lax.dot_general(wv, xb, (((1,), (1,)), ((), ())),
                        preferred_element_type=jnp.float32)
    u = jnp.exp(jnp.tanh(e + b_ref[0]))
    u_ref[0] = u
    uall_ref[pl.ds(bb * NT + t, 1), :] = u
    s1 = lax.dot_general(u, xb, (((1,), (0,)), ((), ())),
                         preferred_element_type=jnp.float32)

    @pl.when(t == 0)
    def _():
        s1_ref[0] = s1

    @pl.when(t != 0)
    def _():
        s1_ref[0] = s1_ref[0] + s1

    @pl.when((bb == B - 1) & (t == NT - 1))
    def _():
        rows = uall_ref[...].reshape(B, NT, TT)
        bits = lax.bitcast_convert_type(rows, jnp.int32)
        z = jnp.sum(rows, axis=(1, 2), keepdims=True)

        def step(_, lohi):
            lo, hi = lohi
            mid = lo + (hi - lo) // 2
            c = jnp.sum(jnp.where(bits >= mid, 1, 0), axis=(1, 2),
                        keepdims=True)
            big = c >= K
            return jnp.where(big, mid, lo), jnp.where(big, hi, mid)

        lo0 = jnp.zeros((B, 1, 1), jnp.int32)
        hi0 = jnp.full((B, 1, 1), 0x7F800000, jnp.int32)
        lo, _ = lax.fori_loop(0, 31, step, (lo0, hi0))
        th = lax.bitcast_convert_type(lo, jnp.float32)
        for b in range(B):
            stats_ref[0, b] = th[b, 0, 0]
            stats_ref[1, b] = 1.0 / z[b, 0, 0]


def _k1(x, wrow, bvec):
    return pl.pallas_call(
        _k1_body,
        grid=(B, NT),
        in_specs=[
            pl.BlockSpec((1, TT, D), lambda b, t: (b, t, 0)),
            pl.BlockSpec((1, D), lambda b, t: (0, 0)),
            pl.BlockSpec(memory_space=pltpu.SMEM),
        ],
        out_specs=[
            pl.BlockSpec((1, 1, TT), lambda b, t: (b, 0, t)),
            pl.BlockSpec((1, 1, D), lambda b, t: (b, 0, 0)),
            pl.BlockSpec((2, B), lambda b, t: (0, 0),
                         memory_space=pltpu.SMEM),
        ],
        out_shape=[
            jax.ShapeDtypeStruct((B, 1, T), jnp.float32),
            jax.ShapeDtypeStruct((B, 1, D), jnp.float32),
            jax.ShapeDtypeStruct((2, B), jnp.float32),
        ],
        scratch_shapes=[pltpu.VMEM((B * NT, TT), jnp.float32)],
    )(x, wrow, bvec)


NC, NS = 2, 16
NW = NC * NS
SPB = NW // B
CH = T // SPB
G = 64
NV = CH // 16


def _k3_sc_body(xf_hbm, uf_hbm, th_hbm, s1_hbm, out_hbm,
                u_v, idx_v, w_v, acc_v, rows0_v, rows1_v, th_v,
                p8_v, fin_v, shared, sem0, sem1):
    sid = lax.axis_index("s")
    wid = lax.axis_index("c") * NS + sid
    b = wid // SPB
    base = b * T + (wid % SPB) * CH

    pltpu.sync_copy(uf_hbm.at[pl.ds(base, CH)], u_v)
    pltpu.sync_copy(th_hbm, th_v.at[pl.ds(0, 32)])
    thb = th_v[pl.ds(b, 16)][0]

    zf = jnp.zeros((16,), jnp.float32)
    zi = jnp.zeros((16,), jnp.int32)
    for j in range(NV + 1):
        idx_v[pl.ds(j * 16, 16)] = zi
        w_v[pl.ds(j * 16, 16)] = zf

    lane = lax.iota(jnp.int32, 16)

    def compact(i, pos):
        uv = u_v[pl.ds(i * 16, 16)]
        mask = uv >= thb
        pc = plsc.cumsum(jnp.where(mask, 1, 0).astype(jnp.int32))
        pos_lane = pos + pc - 1
        plsc.store_scatter(idx_v, [pos_lane], base + i * 16 + lane,
                           mask=mask)
        plsc.store_scatter(w_v, [pos_lane], uv, mask=mask)
        return pos + jnp.max(pc)

    n = lax.fori_loop(0, NV, compact, jnp.int32(0))
    nch = (n + G - 1) // G

    rows = (rows0_v, rows1_v)
    sems = (sem0, sem1)

    def fire(c, k):
        pltpu.async_copy(xf_hbm.at[idx_v.at[pl.ds(c * G, G)]], rows[k],
                         sems[k])

    @pl.when(nch > 0)
    def _():
        fire(0, 0)

    NJ = D // 16
    zacc = tuple(zf for _ in range(NJ))

    def chunk(c, accs):
        k = lax.rem(c, 2)
        @pl.when(c + 1 < nch)
        def _():
            lax.cond(k == 0,
                     lambda: fire(c + 1, 1),
                     lambda: fire(c + 1, 0))

        nrows = jnp.minimum(n - c * G, G)

        def accum(r, accs, rv):
            wq = w_v[pl.ds(c * G + r, 16)][0]
            return tuple(a + wq * rv[r, pl.ds(j * 16, 16)]
                         for j, a in enumerate(accs))

        def body(rv, sem):
            pltpu.make_async_copy(xf_hbm.at[idx_v.at[pl.ds(c * G, G)]],
                                  rv, sem).wait()
            return lax.fori_loop(
                0, nrows, lambda r, a: accum(r, a, rv), accs)

        return lax.cond(k == 0,
                        lambda: body(rows0_v, sem0),
                        lambda: body(rows1_v, sem1))

    accs = lax.fori_loop(0, nch, chunk, zacc)
    for j in range(NJ):
        acc_v[0, pl.ds(j * 16, 16)] = accs[j]

    pltpu.sync_copy(acc_v, shared.at[pl.ds(sid, 1)])
    plsc.subcore_barrier()

    @pl.when(sid % SPB == 0)
    def _():
        lb = sid // SPB
        pltpu.sync_copy(shared.at[pl.ds(lb * SPB, SPB)], p8_v)
        pltpu.sync_copy(s1_hbm.at[pl.ds(b, 1)], fin_v)
        inv = th_v[pl.ds(16 + b, 16)][0]
        for j in range(NJ):
            s2 = p8_v[0, pl.ds(j * 16, 16)]
            for r in range(1, SPB):
                s2 = s2 + p8_v[r, pl.ds(j * 16, 16)]
            fin_v[0, pl.ds(j * 16, 16)] = (
                fin_v[0, pl.ds(j * 16, 16)] + (EMPH - 1.0) * s2) * inv
        pltpu.sync_copy(fin_v, out_hbm.at[pl.ds(b, 1)])


def _k3_sc(stats, x, u, s1):
    xf = x.reshape(B * T, D)
    uf = u.reshape(B * T)
    pad = jnp.zeros((16 - B,), jnp.float32)
    stats32 = jnp.concatenate([stats[0], pad, stats[1], pad])
    mesh = plsc.VectorSubcoreMesh(core_axis_name="c", subcore_axis_name="s")
    f = pl.kernel(
        _k3_sc_body,
        mesh=mesh,
        compiler_params=pltpu.CompilerParams(needs_layout_passes=False),
        out_type=jax.ShapeDtypeStruct((B, D), jnp.float32),
        scratch_types=[
            pltpu.VMEM((CH,), jnp.float32),
            pltpu.VMEM((CH + 16,), jnp.int32),
            pltpu.VMEM((CH + 16,), jnp.float32),
            pltpu.VMEM((1, D), jnp.float32),
            pltpu.VMEM((G, D), jnp.float32),
            pltpu.VMEM((G, D), jnp.float32),
            pltpu.VMEM((48,), jnp.float32),
            pltpu.VMEM((SPB, D), jnp.float32),
            pltpu.VMEM((1, D), jnp.float32),
            pltpu.VMEM_SHARED((NS, D), jnp.float32),
            pltpu.SemaphoreType.DMA,
            pltpu.SemaphoreType.DMA,
        ],
    )
    out = f(xf, uf, stats32, s1.reshape(B, D))
    return out.reshape(B, 1, D)


def kernel(x, W, b):
    wrow = W.reshape(1, D)
    u, s1, stats = _k1(x, wrow, b)
    return _k3_sc(stats, x, u, s1)


# speedup vs baseline: 1.1074x; 1.1074x over previous
import functools

import jax
import jax.numpy as jnp
from jax import lax
from jax.experimental import pallas as pl
from jax.experimental.pallas import tpu as pltpu
from jax.experimental.pallas import tpu_sc as plsc

B, T, D = 4, 8192, 768
K = max(1, T // 10)
EMPH = 1.5
TT = 4096
NT = T // TT


def _k1_body(x_ref, w_ref, b_ref, u_ref, s1_ref, stats_ref, uall_ref):
    bb = pl.program_id(0)
    t = pl.program_id(1)
    xb = x_ref[0]
    wv = w_ref[...]
    e = lax.dot_general(wv, xb, (((1,), (1,)), ((), ())),
                        preferred_element_type=jnp.float32)
    u = jnp.exp(jnp.tanh(e + b_ref[0]))
    u_ref[0] = u
    uall_ref[pl.ds(bb * NT + t, 1), :] = u
    s1 = lax.dot_general(u, xb, (((1,), (0,)), ((), ())),
                         preferred_element_type=jnp.float32)

    @pl.when(t == 0)
    def _():
        s1_ref[0] = s1

    @pl.when(t != 0)
    def _():
        s1_ref[0] = s1_ref[0] + s1

    @pl.when((bb == B - 1) & (t == NT - 1))
    def _():
        rows = uall_ref[...].reshape(B, NT, TT)
        bits = lax.bitcast_convert_type(rows, jnp.int32)
        z = jnp.sum(rows, axis=(1, 2), keepdims=True)

        def step(_, lohi):
            lo, hi = lohi
            mid = lo + (hi - lo) // 2
            c = jnp.sum(jnp.where(bits >= mid, 1, 0), axis=(1, 2),
                        keepdims=True)
            big = c >= K
            return jnp.where(big, mid, lo), jnp.where(big, hi, mid)

        lo0 = jnp.zeros((B, 1, 1), jnp.int32)
        hi0 = jnp.full((B, 1, 1), 0x7F800000, jnp.int32)
        lo, _ = lax.fori_loop(0, 31, step, (lo0, hi0))
        th = lax.bitcast_convert_type(lo, jnp.float32)
        for b in range(B):
            stats_ref[0, b] = th[b, 0, 0]
            stats_ref[1, b] = 1.0 / z[b, 0, 0]


def _k1(x, wrow, bvec):
    return pl.pallas_call(
        _k1_body,
        grid=(B, NT),
        in_specs=[
            pl.BlockSpec((1, TT, D), lambda b, t: (b, t, 0)),
            pl.BlockSpec((1, D), lambda b, t: (0, 0)),
            pl.BlockSpec(memory_space=pltpu.SMEM),
        ],
        out_specs=[
            pl.BlockSpec((1, 1, TT), lambda b, t: (b, 0, t)),
            pl.BlockSpec((1, 1, D), lambda b, t: (b, 0, 0)),
            pl.BlockSpec((2, B), lambda b, t: (0, 0),
                         memory_space=pltpu.SMEM),
        ],
        out_shape=[
            jax.ShapeDtypeStruct((B, 1, T), jnp.float32),
            jax.ShapeDtypeStruct((B, 1, D), jnp.float32),
            jax.ShapeDtypeStruct((2, B), jnp.float32),
        ],
        scratch_shapes=[pltpu.VMEM((B * NT, TT), jnp.float32)],
    )(x, wrow, bvec)


NC, NS = 2, 16
NW = NC * NS
SPB = NW // B
CH = T // SPB
G = 64
NV = CH // 16


def _k3_sc_body(xf_hbm, uf_hbm, th_hbm, out_hbm,
                u_v, idx_v, w_v, acc_v, rows0_v, rows1_v, th_v,
                sem0, sem1):
    wid = lax.axis_index("c") * NS + lax.axis_index("s")
    b = wid // SPB
    base = b * T + (wid % SPB) * CH

    pltpu.sync_copy(uf_hbm.at[pl.ds(base, CH)], u_v)
    pltpu.sync_copy(th_hbm, th_v.at[pl.ds(0, 32)])
    thb = th_v[pl.ds(b, 16)][0]

    zf = jnp.zeros((16,), jnp.float32)
    zi = jnp.zeros((16,), jnp.int32)
    for j in range(NV + 1):
        idx_v[pl.ds(j * 16, 16)] = zi
        w_v[pl.ds(j * 16, 16)] = zf

    lane = lax.iota(jnp.int32, 16)

    def compact(i, pos):
        uv = u_v[pl.ds(i * 16, 16)]
        mask = uv >= thb
        pc = plsc.cumsum(jnp.where(mask, 1, 0).astype(jnp.int32))
        pos_lane = pos + pc - 1
        plsc.store_scatter(idx_v, [pos_lane], base + i * 16 + lane,
                           mask=mask)
        plsc.store_scatter(w_v, [pos_lane], uv, mask=mask)
        return pos + jnp.max(pc)

    n = lax.fori_loop(0, NV, compact, jnp.int32(0))
    nch = (n + G - 1) // G

    rows = (rows0_v, rows1_v)
    sems = (sem0, sem1)

    def fire(c, k):
        pltpu.async_copy(xf_hbm.at[idx_v.at[pl.ds(c * G, G)]], rows[k],
                         sems[k])

    @pl.when(nch > 0)
    def _():
        fire(0, 0)

    NJ = D // 16
    zacc = tuple(zf for _ in range(NJ))

    def chunk(c, accs):
        k = lax.rem(c, 2)
        @pl.when(c + 1 < nch)
        def _():
            lax.cond(k == 0,
                     lambda: fire(c + 1, 1),
                     lambda: fire(c + 1, 0))

        nrows = jnp.minimum(n - c * G, G)

        def accum(r, accs, rv):
            wq = w_v[pl.ds(c * G + r, 16)][0]
            return tuple(a + wq * rv[r, pl.ds(j * 16, 16)]
                         for j, a in enumerate(accs))

        def body(rv, sem):
            pltpu.make_async_copy(xf_hbm.at[idx_v.at[pl.ds(c * G, G)]],
                                  rv, sem).wait()
            return lax.fori_loop(
                0, nrows, lambda r, a: accum(r, a, rv), accs)

        return lax.cond(k == 0,
                        lambda: body(rows0_v, sem0),
                        lambda: body(rows1_v, sem1))

    accs = lax.fori_loop(0, nch, chunk, zacc)
    for j in range(NJ):
        acc_v[0, pl.ds(j * 16, 16)] = accs[j]
    pltpu.sync_copy(acc_v, out_hbm.at[pl.ds(wid, 1)])


def _k3_sc(stats, x, u):
    xf = x.reshape(B * T, D)
    uf = u.reshape(B * T)
    pad = jnp.zeros((16 - B,), jnp.float32)
    stats32 = jnp.concatenate([stats[0], pad, stats[1], pad])
    mesh = plsc.VectorSubcoreMesh(core_axis_name="c", subcore_axis_name="s")
    f = pl.kernel(
        _k3_sc_body,
        mesh=mesh,
        compiler_params=pltpu.CompilerParams(needs_layout_passes=False),
        out_type=jax.ShapeDtypeStruct((NW, D), jnp.float32),
        scratch_types=[
            pltpu.VMEM((CH,), jnp.float32),
            pltpu.VMEM((CH + 16,), jnp.int32),
            pltpu.VMEM((CH + 16,), jnp.float32),
            pltpu.VMEM((1, D), jnp.float32),
            pltpu.VMEM((G, D), jnp.float32),
            pltpu.VMEM((G, D), jnp.float32),
            pltpu.VMEM((48,), jnp.float32),
            pltpu.SemaphoreType.DMA,
            pltpu.SemaphoreType.DMA,
        ],
    )
    return f(xf, uf, stats32)


def _k4_body(stats_ref, s1_ref, p_ref, out_ref):
    for b in range(B):
        invz = stats_ref[1, b]
        s2 = jnp.sum(p_ref[b * SPB:(b + 1) * SPB], axis=0, keepdims=True)
        out_ref[b] = (s1_ref[b] + (EMPH - 1.0) * s2) * invz


def _k4(stats, s1, partials):
    return pl.pallas_call(
        _k4_body,
        in_specs=[
            pl.BlockSpec(memory_space=pltpu.SMEM),
            pl.BlockSpec(memory_space=pltpu.VMEM),
            pl.BlockSpec(memory_space=pltpu.VMEM),
        ],
        out_shape=jax.ShapeDtypeStruct((B, 1, D), jnp.float32),
    )(stats, s1, partials)


def kernel(x, W, b):
    wrow = W.reshape(1, D)
    u, s1, stats = _k1(x, wrow, b)
    partials = _k3_sc(stats, x, u)
    return _k4(stats, s1, partials)


# speedup vs baseline: 1.2577x; 1.1358x over previous
import functools

import jax
import jax.numpy as jnp
from jax import lax
from jax.experimental import pallas as pl
from jax.experimental.pallas import tpu as pltpu
from jax.experimental.pallas import tpu_sc as plsc

B, T, D = 4, 8192, 768
K = max(1, T // 10)
EMPH = 1.5
TT = 4096
NT = T // TT


def _k1_body(x_ref, w_ref, b_ref, u_ref, s1_ref, stats_ref, uall_ref):
    bb = pl.program_id(0)
    t = pl.program_id(1)
    xb = x_ref[0]
    wv = w_ref[...]
    e = lax.dot_general(wv, xb, (((1,), (1,)), ((), ())),
                        preferred_element_type=jnp.float32)
    u = jnp.exp(jnp.tanh(e + b_ref[0]))
    u_ref[0] = u
    uall_ref[pl.ds(bb * NT + t, 1), :] = u
    s1 = lax.dot_general(u, xb, (((1,), (0,)), ((), ())),
                         preferred_element_type=jnp.float32)

    @pl.when(t == 0)
    def _():
        s1_ref[0] = s1

    @pl.when(t != 0)
    def _():
        s1_ref[0] = s1_ref[0] + s1

    @pl.when((bb == B - 1) & (t == NT - 1))
    def _():
        rows = uall_ref[...].reshape(B, NT, TT)
        bits = lax.bitcast_convert_type(rows, jnp.int32)
        z = jnp.sum(rows, axis=(1, 2), keepdims=True)

        def step(_, lohi):
            lo, hi = lohi
            mid = lo + (hi - lo) // 2
            c = jnp.sum(jnp.where(bits >= mid, 1, 0), axis=(1, 2),
                        keepdims=True)
            big = c >= K
            return jnp.where(big, mid, lo), jnp.where(big, hi, mid)

        lo0 = jnp.zeros((B, 1, 1), jnp.int32)
        hi0 = jnp.full((B, 1, 1), 0x7F800000, jnp.int32)
        lo, _ = lax.fori_loop(0, 31, step, (lo0, hi0))
        th = lax.bitcast_convert_type(lo, jnp.float32)
        for b in range(B):
            stats_ref[0, b] = th[b, 0, 0]
            stats_ref[1, b] = 1.0 / z[b, 0, 0]


def _k1(x, wrow, bvec):
    return pl.pallas_call(
        _k1_body,
        grid=(B, NT),
        in_specs=[
            pl.BlockSpec((1, TT, D), lambda b, t: (b, t, 0)),
            pl.BlockSpec((1, D), lambda b, t: (0, 0)),
            pl.BlockSpec(memory_space=pltpu.SMEM),
        ],
        out_specs=[
            pl.BlockSpec((1, 1, TT), lambda b, t: (b, 0, t)),
            pl.BlockSpec((1, 1, D), lambda b, t: (b, 0, 0)),
            pl.BlockSpec((2, B), lambda b, t: (0, 0),
                         memory_space=pltpu.SMEM),
        ],
        out_shape=[
            jax.ShapeDtypeStruct((B, 1, T), jnp.float32),
            jax.ShapeDtypeStruct((B, 1, D), jnp.float32),
            jax.ShapeDtypeStruct((2, B), jnp.float32),
        ],
        scratch_shapes=[pltpu.VMEM((B * NT, TT), jnp.float32)],
        compiler_params=pltpu.CompilerParams(
            dimension_semantics=("arbitrary", "arbitrary")),
    )(x, wrow, bvec)


NC, NS = 2, 16
NW = NC * NS
SPB = NW // B
CH = T // SPB
G = 32
NV = CH // 16


def _k3_sc_body(xf_hbm, uf_hbm, th_hbm, out_hbm,
                u_v, idx_v, w_v, acc_v, rows0_v, rows1_v, th_v,
                sem0, sem1):
    wid = lax.axis_index("c") * NS + lax.axis_index("s")
    b = wid // SPB
    base = b * T + (wid % SPB) * CH

    pltpu.sync_copy(uf_hbm.at[pl.ds(base, CH)], u_v)
    pltpu.sync_copy(th_hbm, th_v.at[pl.ds(0, 32)])
    thb = th_v[pl.ds(b, 16)][0]

    zf = jnp.zeros((16,), jnp.float32)
    zi = jnp.zeros((16,), jnp.int32)
    for j in range(NV + 1):
        idx_v[pl.ds(j * 16, 16)] = zi
        w_v[pl.ds(j * 16, 16)] = zf

    lane = lax.iota(jnp.int32, 16)

    def compact(i, pos):
        uv = u_v[pl.ds(i * 16, 16)]
        mask = uv >= thb
        pc = plsc.cumsum(jnp.where(mask, 1, 0).astype(jnp.int32))
        pos_lane = pos + pc - 1
        plsc.store_scatter(idx_v, [pos_lane], base + i * 16 + lane,
                           mask=mask)
        plsc.store_scatter(w_v, [pos_lane], uv, mask=mask)
        return pos + jnp.max(pc)

    n = lax.fori_loop(0, NV, compact, jnp.int32(0))
    nch = (n + G - 1) // G

    rows = (rows0_v, rows1_v)
    sems = (sem0, sem1)

    def fire(c, k):
        pltpu.async_copy(xf_hbm.at[idx_v.at[pl.ds(c * G, G)]], rows[k],
                         sems[k])

    @pl.when(nch > 0)
    def _():
        fire(0, 0)

    NJ = D // 16
    zacc = tuple(zf for _ in range(NJ))

    def chunk(c, accs):
        k = lax.rem(c, 2)
        @pl.when(c + 1 < nch)
        def _():
            lax.cond(k == 0,
                     lambda: fire(c + 1, 1),
                     lambda: fire(c + 1, 0))

        nrows = jnp.minimum(n - c * G, G)

        def accum(r, accs, rv):
            wq = w_v[pl.ds(c * G + r, 16)][0]
            return tuple(a + wq * rv[r, pl.ds(j * 16, 16)]
                         for j, a in enumerate(accs))

        def body(rv, sem):
            pltpu.make_async_copy(xf_hbm.at[idx_v.at[pl.ds(c * G, G)]],
                                  rv, sem).wait()
            return lax.fori_loop(
                0, nrows, lambda r, a: accum(r, a, rv), accs)

        return lax.cond(k == 0,
                        lambda: body(rows0_v, sem0),
                        lambda: body(rows1_v, sem1))

    accs = lax.fori_loop(0, nch, chunk, zacc)
    for j in range(NJ):
        acc_v[0, pl.ds(j * 16, 16)] = accs[j]
    pltpu.sync_copy(acc_v, out_hbm.at[pl.ds(wid, 1)])


def _k3_sc(stats, x, u):
    xf = x.reshape(B * T, D)
    uf = u.reshape(B * T)
    pad = jnp.zeros((16 - B,), jnp.float32)
    stats32 = jnp.concatenate([stats[0], pad, stats[1], pad])
    mesh = plsc.VectorSubcoreMesh(core_axis_name="c", subcore_axis_name="s")
    f = pl.kernel(
        _k3_sc_body,
        mesh=mesh,
        compiler_params=pltpu.CompilerParams(needs_layout_passes=False),
        out_type=jax.ShapeDtypeStruct((NW, D), jnp.float32),
        scratch_types=[
            pltpu.VMEM((CH,), jnp.float32),
            pltpu.VMEM((CH + 16,), jnp.int32),
            pltpu.VMEM((CH + 16,), jnp.float32),
            pltpu.VMEM((1, D), jnp.float32),
            pltpu.VMEM((G, D), jnp.float32),
            pltpu.VMEM((G, D), jnp.float32),
            pltpu.VMEM((48,), jnp.float32),
            pltpu.SemaphoreType.DMA,
            pltpu.SemaphoreType.DMA,
        ],
    )
    return f(xf, uf, stats32)


def _k4_body(stats_ref, s1_ref, p_ref, out_ref):
    for b in range(B):
        invz = stats_ref[1, b]
        s2 = jnp.sum(p_ref[b * SPB:(b + 1) * SPB], axis=0, keepdims=True)
        out_ref[b] = (s1_ref[b] + (EMPH - 1.0) * s2) * invz


def _k4(stats, s1, partials):
    return pl.pallas_call(
        _k4_body,
        in_specs=[
            pl.BlockSpec(memory_space=pltpu.SMEM),
            pl.BlockSpec(memory_space=pltpu.VMEM),
            pl.BlockSpec(memory_space=pltpu.VMEM),
        ],
        out_shape=jax.ShapeDtypeStruct((B, 1, D), jnp.float32),
    )(stats, s1, partials)


def kernel(x, W, b):
    wrow = W.reshape(1, D)
    u, s1, stats = _k1(x, wrow, b)
    partials = _k3_sc(stats, x, u)
    return _k4(stats, s1, partials)


# speedup vs baseline: 1.4129x; 1.1234x over previous
import functools

import jax
import jax.numpy as jnp
from jax import lax
from jax.experimental import pallas as pl
from jax.experimental.pallas import tpu as pltpu
from jax.experimental.pallas import tpu_sc as plsc

B, T, D = 4, 8192, 768
K = max(1, T // 10)
EMPH = 1.5
TT = 4096
NT = T // TT


def _k1_body(x_ref, w_ref, b_ref, u_ref, s1_ref, stats_ref, uall_ref):
    bb = pl.program_id(0)
    t = pl.program_id(1)
    xb = x_ref[0]
    wv = w_ref[...]
    e = lax.dot_general(wv, xb, (((1,), (1,)), ((), ())),
                        preferred_element_type=jnp.float32)
    u = jnp.exp(jnp.tanh(e + b_ref[0]))
    u_ref[0] = u
    uall_ref[pl.ds(bb * NT + t, 1), :] = u
    s1 = lax.dot_general(u, xb, (((1,), (0,)), ((), ())),
                         preferred_element_type=jnp.float32)

    @pl.when(t == 0)
    def _():
        s1_ref[0] = s1

    @pl.when(t != 0)
    def _():
        s1_ref[0] = s1_ref[0] + s1

    @pl.when((bb == B - 1) & (t == NT - 1))
    def _():
        rows = uall_ref[...].reshape(B, NT, TT)
        bits = lax.bitcast_convert_type(rows, jnp.int32)
        z = jnp.sum(rows, axis=(1, 2), keepdims=True)

        def step(_, lohi):
            lo, hi = lohi
            mid = lo + (hi - lo) // 2
            c = jnp.sum(jnp.where(bits >= mid, 1, 0), axis=(1, 2),
                        keepdims=True)
            big = c >= K
            return jnp.where(big, mid, lo), jnp.where(big, hi, mid)

        lo0 = jnp.zeros((B, 1, 1), jnp.int32)
        hi0 = jnp.full((B, 1, 1), 0x7F800000, jnp.int32)
        lo, _ = lax.fori_loop(0, 31, step, (lo0, hi0))
        th = lax.bitcast_convert_type(lo, jnp.float32)
        for b in range(B):
            stats_ref[0, b] = th[b, 0, 0]
            stats_ref[1, b] = 1.0 / z[b, 0, 0]


def _k1(x, wrow, bvec):
    return pl.pallas_call(
        _k1_body,
        grid=(B, NT),
        in_specs=[
            pl.BlockSpec((1, TT, D), lambda b, t: (b, t, 0)),
            pl.BlockSpec((1, D), lambda b, t: (0, 0)),
            pl.BlockSpec(memory_space=pltpu.SMEM),
        ],
        out_specs=[
            pl.BlockSpec((1, 1, TT), lambda b, t: (b, 0, t)),
            pl.BlockSpec((1, 1, D), lambda b, t: (b, 0, 0)),
            pl.BlockSpec((2, B), lambda b, t: (0, 0),
                         memory_space=pltpu.SMEM),
        ],
        out_shape=[
            jax.ShapeDtypeStruct((B, 1, T), jnp.float32),
            jax.ShapeDtypeStruct((B, 1, D), jnp.float32),
            jax.ShapeDtypeStruct((2, B), jnp.float32),
        ],
        scratch_shapes=[pltpu.VMEM((B * NT, TT), jnp.float32)],
        compiler_params=pltpu.CompilerParams(
            dimension_semantics=("arbitrary", "arbitrary")),
    )(x, wrow, bvec)


NC, NS = 2, 16
NW = NC * NS
SPB = NW // B
CH = T // SPB
G = 16
NV = CH // 16


def _k3_sc_body(xf_hbm, uf_hbm, th_hbm, out_hbm,
                u_v, idx_v, w_v, acc_v, rows0_v, rows1_v, th_v,
                sem0, sem1):
    wid = lax.axis_index("c") * NS + lax.axis_index("s")
    b = wid // SPB
    base = b * T + (wid % SPB) * CH

    pltpu.sync_copy(uf_hbm.at[pl.ds(base, CH)], u_v)
    pltpu.sync_copy(th_hbm, th_v.at[pl.ds(0, 32)])
    thb = th_v[pl.ds(b, 16)][0]

    zf = jnp.zeros((16,), jnp.float32)
    zi = jnp.zeros((16,), jnp.int32)
    for j in range(NV + 1):
        idx_v[pl.ds(j * 16, 16)] = zi
        w_v[pl.ds(j * 16, 16)] = zf

    lane = lax.iota(jnp.int32, 16)

    def compact(i, pos):
        uv = u_v[pl.ds(i * 16, 16)]
        mask = uv >= thb
        pc = plsc.cumsum(jnp.where(mask, 1, 0).astype(jnp.int32))
        pos_lane = pos + pc - 1
        plsc.store_scatter(idx_v, [pos_lane], base + i * 16 + lane,
                           mask=mask)
        plsc.store_scatter(w_v, [pos_lane], uv, mask=mask)
        return pos + jnp.max(pc)

    n = lax.fori_loop(0, NV, compact, jnp.int32(0))
    nch = (n + G - 1) // G

    rows = (rows0_v, rows1_v)
    sems = (sem0, sem1)

    def fire(c, k):
        pltpu.async_copy(xf_hbm.at[idx_v.at[pl.ds(c * G, G)]], rows[k],
                         sems[k])

    @pl.when(nch > 0)
    def _():
        fire(0, 0)

    NJ = D // 16
    zacc = tuple(zf for _ in range(NJ))

    def chunk(c, accs):
        k = lax.rem(c, 2)
        @pl.when(c + 1 < nch)
        def _():
            lax.cond(k == 0,
                     lambda: fire(c + 1, 1),
                     lambda: fire(c + 1, 0))

        nrows = jnp.minimum(n - c * G, G)

        def accum(r, accs, rv):
            wq = w_v[pl.ds(c * G + r, 16)][0]
            return tuple(a + wq * rv[r, pl.ds(j * 16, 16)]
                         for j, a in enumerate(accs))

        def body(rv, sem):
            pltpu.make_async_copy(xf_hbm.at[idx_v.at[pl.ds(c * G, G)]],
                                  rv, sem).wait()
            return lax.fori_loop(
                0, nrows, lambda r, a: accum(r, a, rv), accs)

        return lax.cond(k == 0,
                        lambda: body(rows0_v, sem0),
                        lambda: body(rows1_v, sem1))

    accs = lax.fori_loop(0, nch, chunk, zacc)
    for j in range(NJ):
        acc_v[0, pl.ds(j * 16, 16)] = accs[j]
    pltpu.sync_copy(acc_v, out_hbm.at[pl.ds(wid, 1)])


def _k3_sc(stats, x, u):
    xf = x.reshape(B * T, D)
    uf = u.reshape(B * T)
    pad = jnp.zeros((16 - B,), jnp.float32)
    stats32 = jnp.concatenate([stats[0], pad, stats[1], pad])
    mesh = plsc.VectorSubcoreMesh(core_axis_name="c", subcore_axis_name="s")
    f = pl.kernel(
        _k3_sc_body,
        mesh=mesh,
        compiler_params=pltpu.CompilerParams(needs_layout_passes=False),
        out_type=jax.ShapeDtypeStruct((NW, D), jnp.float32),
        scratch_types=[
            pltpu.VMEM((CH,), jnp.float32),
            pltpu.VMEM((CH + 16,), jnp.int32),
            pltpu.VMEM((CH + 16,), jnp.float32),
            pltpu.VMEM((1, D), jnp.float32),
            pltpu.VMEM((G, D), jnp.float32),
            pltpu.VMEM((G, D), jnp.float32),
            pltpu.VMEM((48,), jnp.float32),
            pltpu.SemaphoreType.DMA,
            pltpu.SemaphoreType.DMA,
        ],
    )
    return f(xf, uf, stats32)


def _k4_body(stats_ref, s1_ref, p_ref, out_ref):
    for b in range(B):
        invz = stats_ref[1, b]
        s2 = jnp.sum(p_ref[b * SPB:(b + 1) * SPB], axis=0, keepdims=True)
        out_ref[b] = (s1_ref[b] + (EMPH - 1.0) * s2) * invz


def _k4(stats, s1, partials):
    return pl.pallas_call(
        _k4_body,
        in_specs=[
            pl.BlockSpec(memory_space=pltpu.SMEM),
            pl.BlockSpec(memory_space=pltpu.VMEM),
            pl.BlockSpec(memory_space=pltpu.VMEM),
        ],
        out_shape=jax.ShapeDtypeStruct((B, 1, D), jnp.float32),
    )(stats, s1, partials)


def kernel(x, W, b):
    wrow = W.reshape(1, D)
    u, s1, stats = _k1(x, wrow, b)
    partials = _k3_sc(stats, x, u)
    return _k4(stats, s1, partials)


# speedup vs baseline: 1.4819x; 1.0489x over previous
import functools

import jax
import jax.numpy as jnp
from jax import lax
from jax.experimental import pallas as pl
from jax.experimental.pallas import tpu as pltpu
from jax.experimental.pallas import tpu_sc as plsc

B, T, D = 4, 8192, 768
K = max(1, T // 10)
EMPH = 1.5
TT = 4096
NT = T // TT


def _k1_body(x_ref, w_ref, b_ref, u_ref, s1_ref, stats_ref, uall_ref):
    bb = pl.program_id(0)
    t = pl.program_id(1)
    xb = x_ref[0]
    wv = w_ref[...]
    e = lax.dot_general(wv, xb, (((1,), (1,)), ((), ())),
                        preferred_element_type=jnp.float32)
    u = jnp.exp(jnp.tanh(e + b_ref[0]))
    u_ref[0] = u
    uall_ref[pl.ds(bb * NT + t, 1), :] = u
    s1 = lax.dot_general(u, xb, (((1,), (0,)), ((), ())),
                         preferred_element_type=jnp.float32)

    @pl.when(t == 0)
    def _():
        s1_ref[0] = s1

    @pl.when(t != 0)
    def _():
        s1_ref[0] = s1_ref[0] + s1

    @pl.when((bb == B - 1) & (t == NT - 1))
    def _():
        rows = uall_ref[...].reshape(B, NT, TT)
        bits = lax.bitcast_convert_type(rows, jnp.int32)
        z = jnp.sum(rows, axis=(1, 2), keepdims=True)

        def step(_, lohi):
            lo, hi = lohi
            mid = lo + (hi - lo) // 2
            c = jnp.sum(jnp.where(bits >= mid, 1, 0), axis=(1, 2),
                        keepdims=True)
            big = c >= K
            return jnp.where(big, mid, lo), jnp.where(big, hi, mid)

        lo0 = jnp.zeros((B, 1, 1), jnp.int32)
        hi0 = jnp.full((B, 1, 1), 0x7F800000, jnp.int32)
        lo, _ = lax.fori_loop(0, 31, step, (lo0, hi0))
        th = lax.bitcast_convert_type(lo, jnp.float32)
        for b in range(B):
            stats_ref[0, b] = th[b, 0, 0]
            stats_ref[1, b] = 1.0 / z[b, 0, 0]


def _k1(x, wrow, bvec):
    return pl.pallas_call(
        _k1_body,
        grid=(B, NT),
        in_specs=[
            pl.BlockSpec((1, TT, D), lambda b, t: (b, t, 0)),
            pl.BlockSpec((1, D), lambda b, t: (0, 0)),
            pl.BlockSpec(memory_space=pltpu.SMEM),
        ],
        out_specs=[
            pl.BlockSpec((1, 1, TT), lambda b, t: (b, 0, t)),
            pl.BlockSpec((1, 1, D), lambda b, t: (b, 0, 0)),
            pl.BlockSpec((2, B), lambda b, t: (0, 0),
                         memory_space=pltpu.SMEM),
        ],
        out_shape=[
            jax.ShapeDtypeStruct((B, 1, T), jnp.float32),
            jax.ShapeDtypeStruct((B, 1, D), jnp.float32),
            jax.ShapeDtypeStruct((2, B), jnp.float32),
        ],
        scratch_shapes=[pltpu.VMEM((B * NT, TT), jnp.float32)],
        compiler_params=pltpu.CompilerParams(
            dimension_semantics=("arbitrary", "arbitrary")),
    )(x, wrow, bvec)


NC, NS = 2, 16
NW = NC * NS
SPB = NW // B
CH = T // SPB
G = 8
NV = CH // 16


def _k3_sc_body(xf_hbm, uf_hbm, th_hbm, out_hbm,
                u_v, idx_v, w_v, acc_v, rows0_v, rows1_v, th_v,
                sem0, sem1):
    wid = lax.axis_index("c") * NS + lax.axis_index("s")
    b = wid // SPB
    base = b * T + (wid % SPB) * CH

    pltpu.sync_copy(uf_hbm.at[pl.ds(base, CH)], u_v)
    pltpu.sync_copy(th_hbm, th_v.at[pl.ds(0, 32)])
    thb = th_v[pl.ds(b, 16)][0]

    zf = jnp.zeros((16,), jnp.float32)
    zi = jnp.zeros((16,), jnp.int32)
    for j in range(NV + 1):
        idx_v[pl.ds(j * 16, 16)] = zi
        w_v[pl.ds(j * 16, 16)] = zf

    lane = lax.iota(jnp.int32, 16)

    def compact(i, pos):
        uv = u_v[pl.ds(i * 16, 16)]
        mask = uv >= thb
        pc = plsc.cumsum(jnp.where(mask, 1, 0).astype(jnp.int32))
        pos_lane = pos + pc - 1
        plsc.store_scatter(idx_v, [pos_lane], base + i * 16 + lane,
                           mask=mask)
        plsc.store_scatter(w_v, [pos_lane], uv, mask=mask)
        return pos + jnp.max(pc)

    n = lax.fori_loop(0, NV, compact, jnp.int32(0))
    nch = (n + G - 1) // G

    rows = (rows0_v, rows1_v)
    sems = (sem0, sem1)

    def fire(c, k):
        pltpu.async_copy(xf_hbm.at[idx_v.at[pl.ds(c * G, G)]], rows[k],
                         sems[k])

    @pl.when(nch > 0)
    def _():
        fire(0, 0)

    NJ = D // 16
    zacc = tuple(zf for _ in range(NJ))

    def chunk(c, accs):
        k = lax.rem(c, 2)
        @pl.when(c + 1 < nch)
        def _():
            lax.cond(k == 0,
                     lambda: fire(c + 1, 1),
                     lambda: fire(c + 1, 0))

        nrows = jnp.minimum(n - c * G, G)

        def accum(r, accs, rv):
            wq = w_v[pl.ds(c * G + r, 16)][0]
            return tuple(a + wq * rv[r, pl.ds(j * 16, 16)]
                         for j, a in enumerate(accs))

        def body(rv, sem):
            pltpu.make_async_copy(xf_hbm.at[idx_v.at[pl.ds(c * G, G)]],
                                  rv, sem).wait()
            return lax.fori_loop(
                0, nrows, lambda r, a: accum(r, a, rv), accs)

        return lax.cond(k == 0,
                        lambda: body(rows0_v, sem0),
                        lambda: body(rows1_v, sem1))

    accs = lax.fori_loop(0, nch, chunk, zacc)
    for j in range(NJ):
        acc_v[0, pl.ds(j * 16, 16)] = accs[j]
    pltpu.sync_copy(acc_v, out_hbm.at[pl.ds(wid, 1)])


def _k3_sc(stats, x, u):
    xf = x.reshape(B * T, D)
    uf = u.reshape(B * T)
    pad = jnp.zeros((16 - B,), jnp.float32)
    stats32 = jnp.concatenate([stats[0], pad, stats[1], pad])
    mesh = plsc.VectorSubcoreMesh(core_axis_name="c", subcore_axis_name="s")
    f = pl.kernel(
        _k3_sc_body,
        mesh=mesh,
        compiler_params=pltpu.CompilerParams(needs_layout_passes=False),
        out_type=jax.ShapeDtypeStruct((NW, D), jnp.float32),
        scratch_types=[
            pltpu.VMEM((CH,), jnp.float32),
            pltpu.VMEM((CH + 16,), jnp.int32),
            pltpu.VMEM((CH + 16,), jnp.float32),
            pltpu.VMEM((1, D), jnp.float32),
            pltpu.VMEM((G, D), jnp.float32),
            pltpu.VMEM((G, D), jnp.float32),
            pltpu.VMEM((48,), jnp.float32),
            pltpu.SemaphoreType.DMA,
            pltpu.SemaphoreType.DMA,
        ],
    )
    return f(xf, uf, stats32)


def _k4_body(stats_ref, s1_ref, p_ref, out_ref):
    for b in range(B):
        invz = stats_ref[1, b]
        s2 = jnp.sum(p_ref[b * SPB:(b + 1) * SPB], axis=0, keepdims=True)
        out_ref[b] = (s1_ref[b] + (EMPH - 1.0) * s2) * invz


def _k4(stats, s1, partials):
    return pl.pallas_call(
        _k4_body,
        in_specs=[
            pl.BlockSpec(memory_space=pltpu.SMEM),
            pl.BlockSpec(memory_space=pltpu.VMEM),
            pl.BlockSpec(memory_space=pltpu.VMEM),
        ],
        out_shape=jax.ShapeDtypeStruct((B, 1, D), jnp.float32),
    )(stats, s1, partials)


def kernel(x, W, b):
    wrow = W.reshape(1, D)
    u, s1, stats = _k1(x, wrow, b)
    partials = _k3_sc(stats, x, u)
    return _k4(stats, s1, partials)
